# Initial kernel scaffold; baseline (speedup 1.0000x reference)
import jax
import jax.numpy as jnp
from jax.experimental import pallas as pl


def kernel(x, edge_index, edge_attr, node_batch, params):
    raise NotImplementedError("write your pallas kernel here")


# speedup vs baseline: 2.1372x; 2.1372x over previous
import functools
import math

import jax
import jax.numpy as jnp
from jax import lax
from jax.experimental import pallas as pl
from jax.experimental.pallas import tpu as pltpu
from jax.experimental.pallas import tpu_sc as plsc

NN = 10000
NE = 160000
NG = 64
NCC = 2
NSC = 16
NW = NCC * NSC
CE = 32
CR = 16
L = 16

f32 = jnp.float32
i32 = jnp.int32


def _qkvs_call(x, Wc, bc, C):
    NNr, Kp = x.shape
    BR = 1000

    def body(x_ref, w_ref, b_ref, qo, ko, vo, so):
        acc = jnp.dot(x_ref[...], w_ref[...], preferred_element_type=f32)
        acc = acc + b_ref[...]
        qo[...] = acc[:, :C]
        ko[...] = acc[:, C:2 * C]
        vo[...] = acc[:, 2 * C:3 * C]
        so[...] = acc[:, 3 * C:]

    return pl.pallas_call(
        body,
        grid=(NNr // BR,),
        in_specs=[
            pl.BlockSpec((BR, Kp), lambda i: (i, 0)),
            pl.BlockSpec((Kp, 4 * C), lambda i: (0, 0)),
            pl.BlockSpec((1, 4 * C), lambda i: (0, 0)),
        ],
        out_specs=[pl.BlockSpec((BR, C), lambda i: (i, 0))] * 4,
        out_shape=[jax.ShapeDtypeStruct((NNr, C), f32)] * 4,
    )(x, Wc, bc)


def _e_call(ea16, W1, W2, W3):
    BR = 4000

    def body(a_ref, w1, w2, w3, o1, o2, o3):
        a = a_ref[...]
        o1[...] = jnp.dot(a, w1[...], preferred_element_type=f32)
        o2[...] = jnp.dot(a, w2[...], preferred_element_type=f32)
        o3[...] = jnp.dot(a, w3[...], preferred_element_type=f32)

    return pl.pallas_call(
        body,
        grid=(NE // BR,),
        in_specs=[
            pl.BlockSpec((BR, 16), lambda i: (i, 0)),
            pl.BlockSpec((16, 256), lambda i: (0, 0)),
            pl.BlockSpec((16, 512), lambda i: (0, 0)),
            pl.BlockSpec((16, 512), lambda i: (0, 0)),
        ],
        out_specs=[
            pl.BlockSpec((BR, 256), lambda i: (i, 0)),
            pl.BlockSpec((BR, 512), lambda i: (i, 0)),
            pl.BlockSpec((BR, 512), lambda i: (i, 0)),
        ],
        out_shape=[
            jax.ShapeDtypeStruct((NE, 256), f32),
            jax.ShapeDtypeStruct((NE, 512), f32),
            jax.ShapeDtypeStruct((NE, 512), f32),
        ],
    )(ea16, W1, W2, W3)


def _combine_call(m, s):
    NNr, C = m.shape
    BR = 1000

    def body(m_ref, s_ref, o_ref, h_ref):
        o = m_ref[...] + s_ref[...]
        o_ref[...] = o
        h_ref[...] = jnp.where(o >= 0, o, 0.2 * o)

    return pl.pallas_call(
        body,
        grid=(NNr // BR,),
        in_specs=[pl.BlockSpec((BR, C), lambda i: (i, 0))] * 2,
        out_specs=[pl.BlockSpec((BR, C), lambda i: (i, 0))] * 2,
        out_shape=[jax.ShapeDtypeStruct((NNr, C), f32)] * 2,
    )(m, s)


def _sread(vref, i):
    return vref[pl.ds(i, L)][0]


def _attn_call(q, k, v, es, srcs, dsts, ebnd, C):
    nch = C // L
    inv_sqrt = 1.0 / math.sqrt(C)
    mesh = plsc.VectorSubcoreMesh(core_axis_name="c", subcore_axis_name="s")

    @functools.partial(
        pl.kernel,
        out_type=jax.ShapeDtypeStruct((NN, C), f32),
        mesh=mesh,
        scratch_types=[
            pltpu.VMEM((CE + L,), i32),
            pltpu.VMEM((CE + L,), i32),
            pltpu.VMEM((CE, C), f32),
            pltpu.VMEM((CE, C), f32),
            pltpu.VMEM((CE, C), f32),
            pltpu.VMEM((CE, C), f32),
            pltpu.VMEM((C,), f32),
            pltpu.VMEM((L,), f32),
            pltpu.VMEM((C,), f32),
            pltpu.VMEM((C,), f32),
            pltpu.VMEM((48,), i32),
            pltpu.SMEM((8,), i32),
            pltpu.SemaphoreType.DMA,
            pltpu.SemaphoreType.DMA,
        ],
        compiler_params=pltpu.CompilerParams(needs_layout_passes=False),
    )
    def attn(q_hbm, k_hbm, v_hbm, e_hbm, src_hbm, dst_hbm, ebnd_hbm, out_hbm,
             idxs_v, idxd_v, qb, kb, vb, eb, sacc, dacc, orow, zrow,
             ebnd_v, cur_s, sem, sem2):
        wid = lax.axis_index("s") * NCC + lax.axis_index("c")
        pltpu.sync_copy(ebnd_hbm, ebnd_v)
        e_lo = _sread(ebnd_v, wid)
        e_hi = _sread(ebnd_v, wid + 1)
        n_lo = (wid * NN) // NW
        n_hi = ((wid + 1) * NN) // NW
        cur_s[0] = -1
        zv = jnp.zeros((L,), f32)
        for c0 in range(nch):
            sacc[pl.ds(c0 * L, L)] = zv
            zrow[pl.ds(c0 * L, L)] = zv
        dacc[...] = zv

        def finalize(cur):
            inv = 1.0 / (dacc[...] + 1e-16)
            for c0 in range(nch):
                sl = pl.ds(c0 * L, L)
                orow[sl] = sacc[sl] * inv
            pltpu.sync_copy(orow, out_hbm.at[cur])

        def zero_fill(zstart, zstop):
            def zf(mr, carry):
                pltpu.sync_copy(zrow, out_hbm.at[mr])
                return carry
            lax.fori_loop(zstart, zstop, zf, 0)

        def chunk_body(ci, carry):
            cbase = ci * CE
            pltpu.sync_copy(src_hbm.at[pl.ds(cbase, CE)], idxs_v.at[pl.ds(0, CE)])
            pltpu.sync_copy(dst_hbm.at[pl.ds(cbase, CE)], idxd_v.at[pl.ds(0, CE)])
            cq = pltpu.async_copy(q_hbm.at[idxd_v.at[pl.ds(0, CE)]], qb, sem)
            ck = pltpu.async_copy(k_hbm.at[idxs_v.at[pl.ds(0, CE)]], kb, sem)
            cv = pltpu.async_copy(v_hbm.at[idxs_v.at[pl.ds(0, CE)]], vb, sem)
            ce = pltpu.async_copy(e_hbm.at[pl.ds(cbase, CE)], eb, sem2)
            cq.wait()
            ck.wait()
            cv.wait()
            ce.wait()
            i_lo = jnp.maximum(e_lo - cbase, 0)
            i_hi = jnp.minimum(e_hi - cbase, CE)

            def edge_body(i, ecarry):
                d_i = _sread(idxd_v, i)
                cur = cur_s[0]

                @pl.when(d_i != cur)
                def _on_change():
                    @pl.when(cur >= 0)
                    def _fin():
                        finalize(cur)
                    zero_fill(jnp.where(cur >= 0, cur + 1, n_lo), d_i)
                    for c0 in range(nch):
                        sacc[pl.ds(c0 * L, L)] = zv
                    dacc[...] = zv
                    cur_s[0] = d_i

                acc = zv
                for c0 in range(nch):
                    sl = pl.ds(c0 * L, L)
                    acc = acc + qb[i, sl] * (kb[i, sl] + eb[i, sl])
                alpha = jnp.sum(acc) * inv_sqrt
                wv = jnp.exp(jnp.full((L,), alpha, f32))
                dacc[...] = dacc[...] + wv
                for c0 in range(nch):
                    sl = pl.ds(c0 * L, L)
                    sacc[sl] = sacc[sl] + wv * (vb[i, sl] + eb[i, sl])
                return ecarry

            lax.fori_loop(i_lo, i_hi, edge_body, 0)
            return carry

        lax.fori_loop(e_lo // CE, (e_hi + CE - 1) // CE, chunk_body, 0)

        cur = cur_s[0]

        @pl.when(cur >= 0)
        def _fin_tail():
            finalize(cur)

        zero_fill(jnp.where(cur >= 0, cur + 1, n_lo), n_hi)

    return attn(q, k, v, es, srcs, dsts, ebnd)


def _pool_call(h, node_batch, nbnd):
    NNr, C = h.shape
    nch = C // L
    mesh = plsc.VectorSubcoreMesh(core_axis_name="c", subcore_axis_name="s")

    @functools.partial(
        pl.kernel,
        out_type=jax.ShapeDtypeStruct((NG, C), f32),
        mesh=mesh,
        scratch_types=[
            pltpu.VMEM((CR, C), f32),
            pltpu.VMEM((CR + L,), i32),
            pltpu.VMEM((C,), f32),
            pltpu.VMEM((C,), f32),
            pltpu.VMEM((C,), f32),
            pltpu.VMEM((48,), i32),
            pltpu.SMEM((8,), i32),
        ],
        compiler_params=pltpu.CompilerParams(needs_layout_passes=False),
    )
    def pool(x_hbm, nb_hbm, nbnd_hbm, out_hbm,
             rb, bbuf, macc, orow, zrow, nbnd_v, cur_s):
        wid = lax.axis_index("s") * NCC + lax.axis_index("c")
        pltpu.sync_copy(nbnd_hbm, nbnd_v)
        n_lo = _sread(nbnd_v, wid)
        n_hi = _sread(nbnd_v, wid + 1)
        g_lo = wid * (NG // NW)
        g_hi = (wid + 1) * (NG // NW)
        cur_s[0] = -1
        zv = jnp.zeros((L,), f32)
        ninf = jnp.full((L,), -jnp.inf, f32)
        for c0 in range(nch):
            macc[pl.ds(c0 * L, L)] = ninf
            zrow[pl.ds(c0 * L, L)] = zv

        def finalize(cur):
            for c0 in range(nch):
                sl = pl.ds(c0 * L, L)
                mv = macc[sl]
                orow[sl] = jnp.where(mv == ninf, zv, mv)
            pltpu.sync_copy(orow, out_hbm.at[cur])

        def zero_fill(zstart, zstop):
            def zf(g, carry):
                pltpu.sync_copy(zrow, out_hbm.at[g])
                return carry
            lax.fori_loop(zstart, zstop, zf, 0)

        def chunk_body(ci, carry):
            cbase = ci * CR
            pltpu.sync_copy(x_hbm.at[pl.ds(cbase, CR)], rb)
            pltpu.sync_copy(nb_hbm.at[pl.ds(cbase, CR)], bbuf.at[pl.ds(0, CR)])
            i_lo = jnp.maximum(n_lo - cbase, 0)
            i_hi = jnp.minimum(n_hi - cbase, CR)

            def row_body(i, rcarry):
                g_i = _sread(bbuf, i)
                cur = cur_s[0]

                @pl.when(g_i != cur)
                def _on_change():
                    @pl.when(cur >= 0)
                    def _fin():
                        finalize(cur)
                    zero_fill(jnp.where(cur >= 0, cur + 1, g_lo), g_i)
                    for c0 in range(nch):
                        macc[pl.ds(c0 * L, L)] = ninf
                    cur_s[0] = g_i

                for c0 in range(nch):
                    sl = pl.ds(c0 * L, L)
                    macc[sl] = jnp.maximum(macc[sl], rb[i, sl])
                return rcarry

            lax.fori_loop(i_lo, i_hi, row_body, 0)
            return carry

        lax.fori_loop(n_lo // CR, (n_hi + CR - 1) // CR, chunk_body, 0)

        cur = cur_s[0]

        @pl.when(cur >= 0)
        def _fin_tail():
            finalize(cur)

        zero_fill(jnp.where(cur >= 0, cur + 1, g_lo), g_hi)

    return pool(h, node_batch, nbnd)


def _layer_weights(p, Kp):
    fi = p['Wq'].shape[0]
    Wc = jnp.concatenate([p['Wq'], p['Wk'], p['Wv'], p['Ws']], axis=1)
    if Kp != fi:
        Wc = jnp.pad(Wc, ((0, Kp - fi), (0, 0)))
    bc = jnp.concatenate([p['bq'], p['bk'], p['bv'], p['bs']])[None, :]
    return Wc, bc


def kernel(x, edge_index, edge_attr, node_batch, params):
    src, dst = edge_index[0], edge_index[1]
    perm = jnp.argsort(dst)
    srcs = jnp.take(src, perm)
    dsts = jnp.take(dst, perm)
    eas = jnp.take(edge_attr, perm, axis=0)
    eas16 = jnp.pad(eas, ((0, 0), (0, 16 - eas.shape[1])))

    node_bnd = (jnp.arange(NW + 1) * NN) // NW
    ebnd = jnp.searchsorted(dsts, node_bnd.astype(i32)).astype(i32)
    ebnd48 = jnp.pad(ebnd, (0, 48 - ebnd.shape[0]))
    nbnd = jnp.searchsorted(node_batch, jnp.arange(0, NG + 1, NG // NW,
                                                   dtype=i32)).astype(i32)
    nbnd48 = jnp.pad(nbnd, (0, 48 - nbnd.shape[0]))

    p1, p2, p3 = params['l1'], params['l2'], params['l3']
    xp = jnp.pad(x, ((0, 0), (0, 128 - x.shape[1])))
    Wc1, bc1 = _layer_weights(p1, 128)
    Wc2, bc2 = _layer_weights(p2, 256)
    Wc3, bc3 = _layer_weights(p3, 512)
    We1 = jnp.pad(p1['We'], ((0, 7), (0, 0)))
    We2 = jnp.pad(p2['We'], ((0, 7), (0, 0)))
    We3 = jnp.pad(p3['We'], ((0, 7), (0, 0)))

    e1, e2, e3 = _e_call(eas16, We1, We2, We3)

    q1, k1, v1, s1 = _qkvs_call(xp, Wc1, bc1, 256)
    m1 = _attn_call(q1, k1, v1, e1, srcs, dsts, ebnd48, 256)
    out1, h1 = _combine_call(m1, s1)

    q2, k2, v2, s2 = _qkvs_call(h1, Wc2, bc2, 512)
    m2 = _attn_call(q2, k2, v2, e2, srcs, dsts, ebnd48, 512)
    _, h2 = _combine_call(m2, s2)

    q3, k3, v3, s3 = _qkvs_call(h2, Wc3, bc3, 512)
    m3 = _attn_call(q3, k3, v3, e3, srcs, dsts, ebnd48, 512)
    out3, _ = _combine_call(m3, s3)

    pooled = _pool_call(out3, node_batch, nbnd48)
    return (out1, pooled, node_batch, edge_index)


# speedup vs baseline: 2.3771x; 1.1123x over previous
import functools
import math

import jax
import jax.numpy as jnp
from jax import lax
from jax.experimental import pallas as pl
from jax.experimental.pallas import tpu as pltpu
from jax.experimental.pallas import tpu_sc as plsc

NN = 10000
NE = 160000
NG = 64
NCC = 2
NSC = 16
NW = NCC * NSC
CE = 16
CR = 16
L = 16

f32 = jnp.float32
i32 = jnp.int32


def _qkvs_call(x, Wc, bc, C):
    NNr, Kp = x.shape
    BR = 1000

    def body(x_ref, w_ref, b_ref, qo, ko, vo, so):
        acc = jnp.dot(x_ref[...], w_ref[...], preferred_element_type=f32)
        acc = acc + b_ref[...]
        qo[...] = acc[:, :C]
        ko[...] = acc[:, C:2 * C]
        vo[...] = acc[:, 2 * C:3 * C]
        so[...] = acc[:, 3 * C:]

    return pl.pallas_call(
        body,
        grid=(NNr // BR,),
        in_specs=[
            pl.BlockSpec((BR, Kp), lambda i: (i, 0)),
            pl.BlockSpec((Kp, 4 * C), lambda i: (0, 0)),
            pl.BlockSpec((1, 4 * C), lambda i: (0, 0)),
        ],
        out_specs=[pl.BlockSpec((BR, C), lambda i: (i, 0))] * 4,
        out_shape=[jax.ShapeDtypeStruct((NNr, C), f32)] * 4,
    )(x, Wc, bc)


def _e_call(ea16, W1, W2, W3):
    BR = 4000

    def body(a_ref, w1, w2, w3, o1, o2, o3):
        a = a_ref[...]
        o1[...] = jnp.dot(a, w1[...], preferred_element_type=f32)
        o2[...] = jnp.dot(a, w2[...], preferred_element_type=f32)
        o3[...] = jnp.dot(a, w3[...], preferred_element_type=f32)

    return pl.pallas_call(
        body,
        grid=(NE // BR,),
        in_specs=[
            pl.BlockSpec((BR, 16), lambda i: (i, 0)),
            pl.BlockSpec((16, 256), lambda i: (0, 0)),
            pl.BlockSpec((16, 512), lambda i: (0, 0)),
            pl.BlockSpec((16, 512), lambda i: (0, 0)),
        ],
        out_specs=[
            pl.BlockSpec((BR, 256), lambda i: (i, 0)),
            pl.BlockSpec((BR, 512), lambda i: (i, 0)),
            pl.BlockSpec((BR, 512), lambda i: (i, 0)),
        ],
        out_shape=[
            jax.ShapeDtypeStruct((NE, 256), f32),
            jax.ShapeDtypeStruct((NE, 512), f32),
            jax.ShapeDtypeStruct((NE, 512), f32),
        ],
    )(ea16, W1, W2, W3)


def _combine_call(m, s):
    NNr, C = m.shape
    BR = 1000

    def body(m_ref, s_ref, o_ref, h_ref):
        o = m_ref[...] + s_ref[...]
        o_ref[...] = o
        h_ref[...] = jnp.where(o >= 0, o, 0.2 * o)

    return pl.pallas_call(
        body,
        grid=(NNr // BR,),
        in_specs=[pl.BlockSpec((BR, C), lambda i: (i, 0))] * 2,
        out_specs=[pl.BlockSpec((BR, C), lambda i: (i, 0))] * 2,
        out_shape=[jax.ShapeDtypeStruct((NNr, C), f32)] * 2,
    )(m, s)


def _sread(vref, i):
    return vref[pl.ds(i, L)][0]


def _attn_call(q, k, v, es, srcs, dsts, ebnd, C):
    nch = C // L
    inv_sqrt = 1.0 / math.sqrt(C)
    mesh = plsc.VectorSubcoreMesh(core_axis_name="c", subcore_axis_name="s")

    @functools.partial(
        pl.kernel,
        out_type=jax.ShapeDtypeStruct((NN, C), f32),
        mesh=mesh,
        scratch_types=[
            pltpu.VMEM((CE + L,), i32),
            pltpu.VMEM((CE + L,), i32),
            pltpu.VMEM((CE + L,), i32),
            pltpu.VMEM((CE + L,), i32),
            pltpu.VMEM((CE, C), f32),
            pltpu.VMEM((CE, C), f32),
            pltpu.VMEM((CE, C), f32),
            pltpu.VMEM((CE, C), f32),
            pltpu.VMEM((CE, C), f32),
            pltpu.VMEM((CE, C), f32),
            pltpu.VMEM((CE, C), f32),
            pltpu.VMEM((CE, C), f32),
            pltpu.VMEM((C,), f32),
            pltpu.VMEM((L,), f32),
            pltpu.VMEM((C,), f32),
            pltpu.VMEM((C,), f32),
            pltpu.VMEM((48,), i32),
            pltpu.SMEM((8,), i32),
            pltpu.SemaphoreType.DMA,
            pltpu.SemaphoreType.DMA,
            pltpu.SemaphoreType.DMA,
        ],
        compiler_params=pltpu.CompilerParams(needs_layout_passes=False),
    )
    def attn(q_hbm, k_hbm, v_hbm, e_hbm, src_hbm, dst_hbm, ebnd_hbm, out_hbm,
             idxsA, idxdA, idxsB, idxdB, qbA, kbA, vbA, ebA, qbB, kbB, vbB,
             ebB, sacc, dacc, orow, zrow, ebnd_v, cur_s, semI, semA, semB):
        wid = lax.axis_index("s") * NCC + lax.axis_index("c")
        pltpu.sync_copy(ebnd_hbm, ebnd_v)
        e_lo = _sread(ebnd_v, wid)
        e_hi = _sread(ebnd_v, wid + 1)
        n_lo = (wid * NN) // NW
        n_hi = ((wid + 1) * NN) // NW
        cur_s[0] = -1
        zv = jnp.zeros((L,), f32)
        for c0 in range(nch):
            sacc[pl.ds(c0 * L, L)] = zv
            zrow[pl.ds(c0 * L, L)] = zv
        dacc[...] = zv

        def finalize(cur):
            inv = 1.0 / (dacc[...] + 1e-16)
            for c0 in range(nch):
                sl = pl.ds(c0 * L, L)
                orow[sl] = sacc[sl] * inv
            pltpu.sync_copy(orow, out_hbm.at[cur])

        def zero_fill(zstart, zstop):
            def zf(mr, carry):
                pltpu.sync_copy(zrow, out_hbm.at[mr])
                return carry
            lax.fori_loop(zstart, zstop, zf, 0)

        def compute_chunk(cbase, idxd, qb, kb, vb, eb):
            i_lo = jnp.maximum(e_lo - cbase, 0)
            i_hi = jnp.minimum(e_hi - cbase, CE)

            def edge_body(i, ecarry):
                d_i = _sread(idxd, i)
                cur = cur_s[0]

                @pl.when(d_i != cur)
                def _on_change():
                    @pl.when(cur >= 0)
                    def _fin():
                        finalize(cur)
                    zero_fill(jnp.where(cur >= 0, cur + 1, n_lo), d_i)
                    for c0 in range(nch):
                        sacc[pl.ds(c0 * L, L)] = zv
                    dacc[...] = zv
                    cur_s[0] = d_i

                acc = zv
                for c0 in range(nch):
                    sl = pl.ds(c0 * L, L)
                    acc = acc + qb[i, sl] * (kb[i, sl] + eb[i, sl])
                alpha = jnp.sum(acc) * inv_sqrt
                wv = jnp.exp(jnp.full((L,), alpha, f32))
                plsc.addupdate(dacc.at[pl.ds(0, L)], wv)
                for c0 in range(nch):
                    sl = pl.ds(c0 * L, L)
                    plsc.addupdate(sacc.at[pl.ds(c0 * L, L)],
                                   wv * (vb[i, sl] + eb[i, sl]))
                return ecarry

            lax.fori_loop(i_lo, i_hi, edge_body, 0)

        nc_lo = e_lo // CE
        nc_hi = (e_hi + CE - 1) // CE
        npairs = (nc_hi - nc_lo + 1) // 2

        def pair_body(cp, carry):
            cb0 = (nc_lo + 2 * cp) * CE
            cb1 = cb0 + CE
            cb1r = jnp.minimum(cb1, NE - CE)
            i1 = pltpu.async_copy(src_hbm.at[pl.ds(cb0, CE)],
                                  idxsA.at[pl.ds(0, CE)], semI)
            i2 = pltpu.async_copy(dst_hbm.at[pl.ds(cb0, CE)],
                                  idxdA.at[pl.ds(0, CE)], semI)
            i3 = pltpu.async_copy(src_hbm.at[pl.ds(cb1r, CE)],
                                  idxsB.at[pl.ds(0, CE)], semI)
            i4 = pltpu.async_copy(dst_hbm.at[pl.ds(cb1r, CE)],
                                  idxdB.at[pl.ds(0, CE)], semI)
            i1.wait(); i2.wait(); i3.wait(); i4.wait()
            a1 = pltpu.async_copy(q_hbm.at[idxdA.at[pl.ds(0, CE)]], qbA, semA)
            a2 = pltpu.async_copy(k_hbm.at[idxsA.at[pl.ds(0, CE)]], kbA, semA)
            a3 = pltpu.async_copy(v_hbm.at[idxsA.at[pl.ds(0, CE)]], vbA, semA)
            a4 = pltpu.async_copy(e_hbm.at[pl.ds(cb0, CE)], ebA, semA)
            b1 = pltpu.async_copy(q_hbm.at[idxdB.at[pl.ds(0, CE)]], qbB, semB)
            b2 = pltpu.async_copy(k_hbm.at[idxsB.at[pl.ds(0, CE)]], kbB, semB)
            b3 = pltpu.async_copy(v_hbm.at[idxsB.at[pl.ds(0, CE)]], vbB, semB)
            b4 = pltpu.async_copy(e_hbm.at[pl.ds(cb1r, CE)], ebB, semB)
            a1.wait(); a2.wait(); a3.wait(); a4.wait()
            compute_chunk(cb0, idxdA, qbA, kbA, vbA, ebA)
            b1.wait(); b2.wait(); b3.wait(); b4.wait()
            compute_chunk(cb1, idxdB, qbB, kbB, vbB, ebB)
            return carry

        lax.fori_loop(0, npairs, pair_body, 0)

        cur = cur_s[0]

        @pl.when(cur >= 0)
        def _fin_tail():
            finalize(cur)

        zero_fill(jnp.where(cur >= 0, cur + 1, n_lo), n_hi)

    return attn(q, k, v, es, srcs, dsts, ebnd)


def _pool_call(h, node_batch, nbnd):
    NNr, C = h.shape
    nch = C // L
    mesh = plsc.VectorSubcoreMesh(core_axis_name="c", subcore_axis_name="s")

    @functools.partial(
        pl.kernel,
        out_type=jax.ShapeDtypeStruct((NG, C), f32),
        mesh=mesh,
        scratch_types=[
            pltpu.VMEM((CR, C), f32),
            pltpu.VMEM((CR + L,), i32),
            pltpu.VMEM((C,), f32),
            pltpu.VMEM((C,), f32),
            pltpu.VMEM((C,), f32),
            pltpu.VMEM((48,), i32),
            pltpu.SMEM((8,), i32),
        ],
        compiler_params=pltpu.CompilerParams(needs_layout_passes=False),
    )
    def pool(x_hbm, nb_hbm, nbnd_hbm, out_hbm,
             rb, bbuf, macc, orow, zrow, nbnd_v, cur_s):
        wid = lax.axis_index("s") * NCC + lax.axis_index("c")
        pltpu.sync_copy(nbnd_hbm, nbnd_v)
        n_lo = _sread(nbnd_v, wid)
        n_hi = _sread(nbnd_v, wid + 1)
        g_lo = wid * (NG // NW)
        g_hi = (wid + 1) * (NG // NW)
        cur_s[0] = -1
        zv = jnp.zeros((L,), f32)
        ninf = jnp.full((L,), -jnp.inf, f32)
        for c0 in range(nch):
            macc[pl.ds(c0 * L, L)] = ninf
            zrow[pl.ds(c0 * L, L)] = zv

        def finalize(cur):
            for c0 in range(nch):
                sl = pl.ds(c0 * L, L)
                mv = macc[sl]
                orow[sl] = jnp.where(mv == ninf, zv, mv)
            pltpu.sync_copy(orow, out_hbm.at[cur])

        def zero_fill(zstart, zstop):
            def zf(g, carry):
                pltpu.sync_copy(zrow, out_hbm.at[g])
                return carry
            lax.fori_loop(zstart, zstop, zf, 0)

        def chunk_body(ci, carry):
            cbase = ci * CR
            pltpu.sync_copy(x_hbm.at[pl.ds(cbase, CR)], rb)
            pltpu.sync_copy(nb_hbm.at[pl.ds(cbase, CR)], bbuf.at[pl.ds(0, CR)])
            i_lo = jnp.maximum(n_lo - cbase, 0)
            i_hi = jnp.minimum(n_hi - cbase, CR)

            def row_body(i, rcarry):
                g_i = _sread(bbuf, i)
                cur = cur_s[0]

                @pl.when(g_i != cur)
                def _on_change():
                    @pl.when(cur >= 0)
                    def _fin():
                        finalize(cur)
                    zero_fill(jnp.where(cur >= 0, cur + 1, g_lo), g_i)
                    for c0 in range(nch):
                        macc[pl.ds(c0 * L, L)] = ninf
                    cur_s[0] = g_i

                for c0 in range(nch):
                    sl = pl.ds(c0 * L, L)
                    macc[sl] = jnp.maximum(macc[sl], rb[i, sl])
                return rcarry

            lax.fori_loop(i_lo, i_hi, row_body, 0)
            return carry

        lax.fori_loop(n_lo // CR, (n_hi + CR - 1) // CR, chunk_body, 0)

        cur = cur_s[0]

        @pl.when(cur >= 0)
        def _fin_tail():
            finalize(cur)

        zero_fill(jnp.where(cur >= 0, cur + 1, g_lo), g_hi)

    return pool(h, node_batch, nbnd)


def _layer_weights(p, Kp):
    fi = p['Wq'].shape[0]
    Wc = jnp.concatenate([p['Wq'], p['Wk'], p['Wv'], p['Ws']], axis=1)
    if Kp != fi:
        Wc = jnp.pad(Wc, ((0, Kp - fi), (0, 0)))
    bc = jnp.concatenate([p['bq'], p['bk'], p['bv'], p['bs']])[None, :]
    return Wc, bc


def kernel(x, edge_index, edge_attr, node_batch, params):
    src, dst = edge_index[0], edge_index[1]
    perm = jnp.argsort(dst)
    srcs = jnp.take(src, perm)
    dsts = jnp.take(dst, perm)
    eas = jnp.take(edge_attr, perm, axis=0)
    eas16 = jnp.pad(eas, ((0, 0), (0, 16 - eas.shape[1])))

    node_bnd = (jnp.arange(NW + 1) * NN) // NW
    ebnd = jnp.searchsorted(dsts, node_bnd.astype(i32)).astype(i32)
    ebnd48 = jnp.pad(ebnd, (0, 48 - ebnd.shape[0]))
    nbnd = jnp.searchsorted(node_batch, jnp.arange(0, NG + 1, NG // NW,
                                                   dtype=i32)).astype(i32)
    nbnd48 = jnp.pad(nbnd, (0, 48 - nbnd.shape[0]))

    p1, p2, p3 = params['l1'], params['l2'], params['l3']
    xp = jnp.pad(x, ((0, 0), (0, 128 - x.shape[1])))
    Wc1, bc1 = _layer_weights(p1, 128)
    Wc2, bc2 = _layer_weights(p2, 256)
    Wc3, bc3 = _layer_weights(p3, 512)
    We1 = jnp.pad(p1['We'], ((0, 7), (0, 0)))
    We2 = jnp.pad(p2['We'], ((0, 7), (0, 0)))
    We3 = jnp.pad(p3['We'], ((0, 7), (0, 0)))

    e1, e2, e3 = _e_call(eas16, We1, We2, We3)

    q1, k1, v1, s1 = _qkvs_call(xp, Wc1, bc1, 256)
    m1 = _attn_call(q1, k1, v1, e1, srcs, dsts, ebnd48, 256)
    out1, h1 = _combine_call(m1, s1)

    q2, k2, v2, s2 = _qkvs_call(h1, Wc2, bc2, 512)
    m2 = _attn_call(q2, k2, v2, e2, srcs, dsts, ebnd48, 512)
    _, h2 = _combine_call(m2, s2)

    q3, k3, v3, s3 = _qkvs_call(h2, Wc3, bc3, 512)
    m3 = _attn_call(q3, k3, v3, e3, srcs, dsts, ebnd48, 512)
    out3, _ = _combine_call(m3, s3)

    pooled = _pool_call(out3, node_batch, nbnd48)
    return (out1, pooled, node_batch, edge_index)
